# Initial kernel scaffold; baseline (speedup 1.0000x reference)
import jax
import jax.numpy as jnp
from jax.experimental import pallas as pl


def kernel(conf_data, reg_data, priors, targets):
    raise NotImplementedError("write your pallas kernel here")


# speedup vs baseline: 48.3709x; 48.3709x over previous
import functools
import math

import jax
import jax.numpy as jnp
from jax import lax
from jax.experimental import pallas as pl
from jax.experimental.pallas import tpu as pltpu

_OMEGA = 10.0
_EPSILON = 2.0
_VAR0 = 0.1
_VAR1 = 0.2
_THRESHOLD = 0.35
_ALPHA = 0.25
_GAMMA = 2.0
_WING_C = _OMEGA - _OMEGA * math.log(1.0 + _OMEGA / _EPSILON)

_P = 16800
_PP = 17408
_G = 64
_B = 32


def _loss_kernel(conf_ref, regt_ref, priors_ref, tgt_ref, out_ref):
    b = pl.program_id(0)

    @pl.when(b == 0)
    def _init():
        out_ref[0] = 0.0
        out_ref[1] = 0.0
        out_ref[2] = 0.0
        out_ref[3] = 0.0
        out_ref[4] = 0.0

    tgt = tgt_ref[0]
    tx1 = tgt[:, 0:1]
    ty1 = tgt[:, 1:2]
    tx2 = tgt[:, 2:3]
    ty2 = tgt[:, 3:4]
    tarea = (tx2 - tx1) * (ty2 - ty1)

    pcx = priors_ref[0:1, :]
    pcy = priors_ref[1:2, :]
    pw = priors_ref[2:3, :]
    ph = priors_ref[3:4, :]
    px1 = pcx - pw * 0.5
    py1 = pcy - ph * 0.5
    px2 = pcx + pw * 0.5
    py2 = pcy + ph * 0.5
    parea = pw * ph

    ix = jnp.maximum(jnp.minimum(tx2, px2) - jnp.maximum(tx1, px1), 0.0)
    iy = jnp.maximum(jnp.minimum(ty2, py2) - jnp.maximum(ty1, py1), 0.0)
    inter = ix * iy
    iou = inter / (tarea + parea - inter)

    pidx = lax.broadcasted_iota(jnp.int32, (1, _PP), 1)
    gidx = lax.broadcasted_iota(jnp.int32, (_G, 1), 0)

    bto = jnp.max(iou, axis=0, keepdims=True)
    bti = jnp.min(jnp.where(iou == bto, gidx, _G), axis=0, keepdims=True)
    rowmax = jnp.max(iou, axis=1, keepdims=True)
    bpi = jnp.min(jnp.where(iou == rowmax, pidx, _PP), axis=1, keepdims=True)

    eq = bpi == pidx
    forced_g = jnp.max(jnp.where(eq, gidx, -1), axis=0, keepdims=True)
    forced = forced_g >= 0
    bti = jnp.where(forced, forced_g, bti)
    bto = jnp.where(forced, 2.0, bto)

    onehot = (gidx == bti).astype(jnp.float32)
    matched = lax.dot_general(
        tgt, onehot, (((0,), (0,)), ((), ())),
        preferred_element_type=jnp.float32)

    lab = matched[18:19, :]
    conf = jnp.where(bto < _THRESHOLD, 0.0, lab)
    pos = conf != 0.0
    pos1 = conf > 0.0
    mpos = pos.astype(jnp.float32)
    mpos1 = pos1.astype(jnp.float32)

    mx1 = matched[0:1, :]
    my1 = matched[1:2, :]
    mx2 = matched[2:3, :]
    my2 = matched[3:4, :]
    g_cx = ((mx1 + mx2) * 0.5 - pcx) / (_VAR0 * pw)
    g_cy = ((my1 + my2) * 0.5 - pcy) / (_VAR0 * ph)
    g_w = jnp.log((mx2 - mx1) / pw) / _VAR1
    g_h = jnp.log((my2 - my1) / ph) / _VAR1
    loc_t = jnp.concatenate([g_cx, g_cy, g_w, g_h], axis=0)

    x = regt_ref[0, 0:4, :] * (1.0 / 192.0)
    sig = jax.nn.sigmoid(x)
    bce = -(loc_t * jnp.log(sig) + (1.0 - loc_t) * jnp.log(1.0 - sig))
    dqf = loc_t - sig
    qfl = dqf * dqf * bce
    qfl_sum = jnp.sum(qfl * mpos)
    n_pos = jnp.sum(mpos)

    lmd = regt_ref[0, 4:18, :] * (1.0 / 192.0)
    mlm = matched[4:18, :]
    pc = jnp.concatenate([pcx, pcy], axis=0)
    pwh = jnp.concatenate([pw, ph], axis=0)
    pc7 = jnp.concatenate([pc] * 7, axis=0)
    pwh7 = jnp.concatenate([pwh] * 7, axis=0)
    lm_t = (mlm - pc7) / (_VAR0 * pwh7)
    d = jnp.abs(lm_t - lmd)
    wing = jnp.where(d < _OMEGA, _OMEGA * jnp.log1p(d * (1.0 / _EPSILON)),
                     d - _WING_C)
    wing_sum = jnp.sum(wing * mpos1)
    n_pos1 = jnp.sum(mpos1)

    valid = (pidx < _P).astype(jnp.float32)
    c = conf_ref[0]
    y = jax.nn.sigmoid(c)
    y_true = mpos
    fl = (-y_true * (1.0 - _ALPHA) * ((1.0 - y) * _GAMMA) * jnp.log(y)
          - (1.0 - y_true) * _ALPHA * (y * y) * jnp.log(1.0 - y))
    fl_sum = jnp.sum(fl * valid)

    out_ref[0] += qfl_sum
    out_ref[1] += n_pos
    out_ref[2] += wing_sum
    out_ref[3] += n_pos1
    out_ref[4] += fl_sum


@jax.jit
def kernel(conf_data, reg_data, priors, targets):
    B, P, _ = conf_data.shape
    pad = _PP - P

    conf_p = jnp.pad(conf_data[:, :, 0], ((0, 0), (0, pad)))
    conf_p = conf_p.reshape(B, 1, _PP)
    regt = jnp.transpose(reg_data, (0, 2, 1))
    regt_p = jnp.pad(regt, ((0, 0), (0, 0), (0, pad)))
    pt = jnp.transpose(priors, (1, 0))
    padvals = jnp.concatenate(
        [jnp.full((2, pad), -10.0, jnp.float32),
         jnp.ones((2, pad), jnp.float32)], axis=0)
    priors_p = jnp.concatenate([pt, padvals], axis=1)

    sums = pl.pallas_call(
        _loss_kernel,
        grid=(B,),
        in_specs=[
            pl.BlockSpec((1, 1, _PP), lambda b: (b, 0, 0)),
            pl.BlockSpec((1, 18, _PP), lambda b: (b, 0, 0)),
            pl.BlockSpec((4, _PP), lambda b: (0, 0)),
            pl.BlockSpec((1, _G, 19), lambda b: (b, 0, 0)),
        ],
        out_specs=pl.BlockSpec(memory_space=pltpu.SMEM),
        out_shape=jax.ShapeDtypeStruct((5,), jnp.float32),
        compiler_params=pltpu.CompilerParams(
            dimension_semantics=("arbitrary",)),
    )(conf_p, regt_p, priors_p, targets)

    qfl_sum, n_pos, wing_sum, n_pos1, fl_sum = (
        sums[0], sums[1], sums[2], sums[3], sums[4])
    loss_l = qfl_sum / jnp.maximum(n_pos * 4.0, 1.0)
    loss_landm = wing_sum / jnp.maximum(n_pos1 * 14.0, 1.0)
    loss_c = fl_sum / (B * P)
    return (loss_l, loss_c, loss_landm)
